# baseline (speedup 1.0000x reference)
import functools

import jax
import jax.numpy as jnp
from jax import lax
from jax.experimental import pallas as pl
from jax.experimental.pallas import tpu as pltpu
from jax.experimental.pallas import tpu_sc as plsc

_NC = 2
_NS = 16
_L = 16
_NW = _NC * _NS
_C = 128


def _sc_body(n, d, nch, zr, x_hbm, src_hbm, dst_hbm, w_hbm, out_hbm,
             acc, rows_a, rows_b, rows_fa, rows_fb,
             srcb_a, srcb_b, dstb_a, dstb_b,
             wb_a, wb_b, semg_a, semg_b, sems_a, sems_b,
             semsrc_a, semsrc_b, semdw_a, semdw_b):
    cid = lax.axis_index("c")
    sid = lax.axis_index("s")
    wid = cid * _NS + sid

    zc = 80
    def zrow(r, carry):
        for j in range(d // _L):
            rows_fa[r, pl.ds(j * _L, _L)] = jnp.zeros((_L,), jnp.float32)
        return carry
    lax.fori_loop(0, zc, zrow, 0)
    nzch = n // zc
    for q in range((nzch + _NS - 1) // _NS):
        idx = sid + _NS * q

        @pl.when(idx < nzch)
        def _():
            pltpu.sync_copy(rows_fa.at[pl.ds(0, zc)],
                            acc.at[pl.ds(idx * zc, zc)])
    plsc.subcore_barrier()

    def load_src(k, srcb, sem):
        pltpu.async_copy(src_hbm.at[wid, k, 0], srcb, sem)

    def wait_src(srcb, sem):
        pltpu.make_async_copy(src_hbm.at[wid, 0, 0], srcb, sem).wait()

    def load_dw(k, dstb, wb, sem):
        pltpu.async_copy(dst_hbm.at[wid, k, 0], dstb, sem)
        pltpu.async_copy(w_hbm.at[wid, k, 0], wb, sem)

    def wait_dw(dstb, wb, sem):
        pltpu.make_async_copy(dst_hbm.at[wid, 0, 0], dstb, sem).wait()
        pltpu.make_async_copy(w_hbm.at[wid, 0, 0], wb, sem).wait()

    def gather(srcb, rows, sem):
        pltpu.async_copy(x_hbm.at[srcb], rows, sem)

    def gather_wait(srcb, rows, sem):
        pltpu.make_async_copy(x_hbm.at[srcb], rows, sem).wait()

    def scatter(dstb, rows, sem):
        pltpu.async_copy(rows, acc.at[dstb], sem, add=True)

    def scatter_wait(dstb, rows, sem):
        pltpu.make_async_copy(rows, acc.at[dstb], sem).wait()

    half = d // 2

    def scale(wb, rows, rows_f):
        def body(i, carry):
            ws = plsc.load_gather(wb, [jnp.full((_L,), i, jnp.int32)])
            for g in range(half // _L):
                packed = plsc.bitcast(rows[i, pl.ds(g * _L, _L)],
                                      jnp.bfloat16)
                lo, hi = plsc.unpack(packed, format=plsc.PackFormat.INTERLEAVED)
                rows_f[i, pl.ds(g * _L, _L)] = lo * ws
                rows_f[i, pl.ds(half + g * _L, _L)] = hi * ws
            return carry
        lax.fori_loop(0, _C, body, 0)

    load_src(0, srcb_a, semsrc_a)
    load_dw(0, dstb_a, wb_a, semdw_a)
    load_src(1, srcb_b, semsrc_b)
    load_dw(1, dstb_b, wb_b, semdw_b)
    wait_src(srcb_a, semsrc_a)
    gather(srcb_a, rows_a, semg_a)
    wait_src(srcb_b, semsrc_b)
    gather(srcb_b, rows_b, semg_b)

    def step(p, carry):
        ka = 2 * p
        gather_wait(srcb_a, rows_a, semg_a)
        load_src(ka + 2, srcb_a, semsrc_a)
        wait_dw(dstb_a, wb_a, semdw_a)
        scale(wb_a, rows_a, rows_fa)
        scatter(dstb_a, rows_fa, sems_a)

        gather_wait(srcb_b, rows_b, semg_b)
        load_src(ka + 3, srcb_b, semsrc_b)
        wait_dw(dstb_b, wb_b, semdw_b)
        scale(wb_b, rows_b, rows_fb)
        scatter(dstb_b, rows_fb, sems_b)

        scatter_wait(dstb_a, rows_fa, sems_a)
        load_dw(ka + 2, dstb_a, wb_a, semdw_a)
        wait_src(srcb_a, semsrc_a)
        gather(srcb_a, rows_a, semg_a)

        scatter_wait(dstb_b, rows_fb, sems_b)
        load_dw(ka + 3, dstb_b, wb_b, semdw_b)
        wait_src(srcb_b, semsrc_b)
        gather(srcb_b, rows_b, semg_b)
        return carry
    lax.fori_loop(0, nch // 2 - 1, step, 0)

    gather_wait(srcb_a, rows_a, semg_a)
    wait_dw(dstb_a, wb_a, semdw_a)
    scale(wb_a, rows_a, rows_fa)
    scatter(dstb_a, rows_fa, sems_a)
    gather_wait(srcb_b, rows_b, semg_b)
    wait_dw(dstb_b, wb_b, semdw_b)
    scale(wb_b, rows_b, rows_fb)
    scatter(dstb_b, rows_fb, sems_b)
    scatter_wait(dstb_a, rows_fa, sems_a)
    scatter_wait(dstb_b, rows_fb, sems_b)
    plsc.subcore_barrier()

    nrch = n // zr
    for q in range((nrch + _NS - 1) // _NS):
        idx = sid + _NS * q

        @pl.when(idx < nrch)
        def _():
            r0 = idx * zr
            pltpu.sync_copy(acc.at[pl.ds(r0, zr)],
                            out_hbm.at[cid, pl.ds(r0, zr)])


def _combine_body(p_ref, o_ref):
    o_ref[...] = p_ref[0] + p_ref[1]


@jax.jit
def kernel(input, edge_index, edge_weight):
    n, d = input.shape
    e = edge_index.shape[1]
    assert e % _NW == 0 and d % _L == 0 and n % 80 == 0
    e_t = e // _NW
    nch = -(-e_t // _C)
    nch += nch % 2
    e_pad = nch * _C

    pad = ((0, 0), (0, e_pad - e_t))
    shape4 = (_NW, nch, 1, _C)
    src = jnp.pad(edge_index[1].reshape(_NW, e_t), pad).reshape(shape4)
    dst = jnp.pad(edge_index[0].reshape(_NW, e_t), pad).reshape(shape4)
    w = jnp.pad(edge_weight.reshape(_NW, e_t), pad).reshape(shape4)

    half = d // 2
    lo = lax.convert_element_type(
        lax.bitcast_convert_type(
            lax.convert_element_type(input[:, :half], jnp.bfloat16),
            jnp.uint16), jnp.uint32)
    hi = lax.convert_element_type(
        lax.bitcast_convert_type(
            lax.convert_element_type(input[:, half:], jnp.bfloat16),
            jnp.uint16), jnp.uint32)
    xp = lax.bitcast_convert_type(lo | (hi << 16), jnp.int32)

    zr = 200
    assert n % zr == 0 and zr % 8 == 0

    mesh = plsc.VectorSubcoreMesh(core_axis_name="c", subcore_axis_name="s",
                                  num_cores=_NC, num_subcores=_NS)
    partial = pl.kernel(
        functools.partial(_sc_body, n, d, nch, zr),
        out_type=jax.ShapeDtypeStruct((_NC, n, d), jnp.float32),
        mesh=mesh,
        compiler_params=pltpu.CompilerParams(needs_layout_passes=False,
                                             use_tc_tiling_on_sc=False),
        scratch_types=[
            pltpu.MemorySpace.VMEM_SHARED((n, d), jnp.float32),
            pltpu.VMEM((_C, half), jnp.int32),
            pltpu.VMEM((_C, half), jnp.int32),
            pltpu.VMEM((_C, d), jnp.float32),
            pltpu.VMEM((_C, d), jnp.float32),
            pltpu.VMEM((_C,), jnp.int32),
            pltpu.VMEM((_C,), jnp.int32),
            pltpu.VMEM((_C,), jnp.int32),
            pltpu.VMEM((_C,), jnp.int32),
            pltpu.VMEM((_C,), jnp.float32),
            pltpu.VMEM((_C,), jnp.float32),
            pltpu.SemaphoreType.DMA,
            pltpu.SemaphoreType.DMA,
            pltpu.SemaphoreType.DMA,
            pltpu.SemaphoreType.DMA,
            pltpu.SemaphoreType.DMA,
            pltpu.SemaphoreType.DMA,
            pltpu.SemaphoreType.DMA,
            pltpu.SemaphoreType.DMA,
        ],
    )(xp, src, dst, w)

    r = 2000
    return pl.pallas_call(
        _combine_body,
        grid=(n // r,),
        in_specs=[pl.BlockSpec((2, r, d), lambda i: (0, i, 0))],
        out_specs=pl.BlockSpec((r, d), lambda i: (i, 0)),
        out_shape=jax.ShapeDtypeStruct((n, d), jnp.float32),
    )(partial)


# speedup vs baseline: 1.4268x; 1.4268x over previous
import functools

import jax
import jax.numpy as jnp
from jax import lax
from jax.experimental import pallas as pl
from jax.experimental.pallas import tpu as pltpu
from jax.experimental.pallas import tpu_sc as plsc

_NC = 2
_NS = 16
_L = 16
_NW = _NC * _NS
_C = 128


def _sc_body(n, d, nch, zr, x_hbm, src_hbm, dst_hbm, w_hbm, out_hbm,
             xs, acc, rows_a, rows_b, srcb_a, srcb_b, dstb_a, dstb_b,
             wb_a, wb_b, semg_a, semg_b, sems_a, sems_b,
             semsrc_a, semsrc_b, semdw_a, semdw_b):
    cid = lax.axis_index("c")
    sid = lax.axis_index("s")
    wid = cid * _NS + sid
    half = d // 2

    def load_src(k, srcb, sem):
        pltpu.async_copy(src_hbm.at[wid, k, 0], srcb, sem)

    def wait_src(srcb, sem):
        pltpu.make_async_copy(src_hbm.at[wid, 0, 0], srcb, sem).wait()

    def load_dw(k, dstb, wb, sem):
        pltpu.async_copy(dst_hbm.at[wid, k, 0], dstb, sem)
        pltpu.async_copy(w_hbm.at[wid, k, 0], wb, sem)

    def wait_dw(dstb, wb, sem):
        pltpu.make_async_copy(dst_hbm.at[wid, 0, 0], dstb, sem).wait()
        pltpu.make_async_copy(w_hbm.at[wid, 0, 0], wb, sem).wait()

    def gather(srcb, rows, sem):
        pltpu.async_copy(xs.at[srcb], rows, sem)

    def gather_wait(srcb, rows, sem):
        pltpu.make_async_copy(xs.at[srcb], rows, sem).wait()

    def scatter(dstb, rows, sem):
        pltpu.async_copy(rows, acc.at[dstb], sem, add=True)

    def scatter_wait(dstb, rows, sem):
        pltpu.make_async_copy(rows, acc.at[dstb], sem).wait()

    def scale(wb, rows):
        def body(i, carry):
            ws = plsc.load_gather(wb, [jnp.full((_L,), i, jnp.int32)])
            for j in range(half // _L):
                rows[i, pl.ds(j * _L, _L)] = rows[i, pl.ds(j * _L, _L)] * ws
            return carry
        lax.fori_loop(0, _C, body, 0)

    zc = 80
    nzch = n // zc
    nrch = n // zr

    def zrow(r, carry):
        for j in range(half // _L):
            rows_a[r, pl.ds(j * _L, _L)] = jnp.zeros((_L,), jnp.float32)
        return carry
    lax.fori_loop(0, zc, zrow, 0)

    for h in range(2):
        for q in range((nzch + _NS - 1) // _NS):
            idx = sid + _NS * q

            @pl.when(idx < nzch)
            def _():
                pltpu.sync_copy(x_hbm.at[h, pl.ds(idx * zc, zc)],
                                xs.at[pl.ds(idx * zc, zc)])
                pltpu.sync_copy(rows_a.at[pl.ds(0, zc)],
                                acc.at[pl.ds(idx * zc, zc)])
        plsc.subcore_barrier()

        load_src(0, srcb_a, semsrc_a)
        load_dw(0, dstb_a, wb_a, semdw_a)
        load_src(1, srcb_b, semsrc_b)
        load_dw(1, dstb_b, wb_b, semdw_b)
        wait_src(srcb_a, semsrc_a)
        gather(srcb_a, rows_a, semg_a)
        wait_src(srcb_b, semsrc_b)
        gather(srcb_b, rows_b, semg_b)

        def step(p, carry):
            ka = 2 * p
            gather_wait(srcb_a, rows_a, semg_a)
            load_src(ka + 2, srcb_a, semsrc_a)
            wait_dw(dstb_a, wb_a, semdw_a)
            scale(wb_a, rows_a)
            scatter(dstb_a, rows_a, sems_a)

            gather_wait(srcb_b, rows_b, semg_b)
            load_src(ka + 3, srcb_b, semsrc_b)
            wait_dw(dstb_b, wb_b, semdw_b)
            scale(wb_b, rows_b)
            scatter(dstb_b, rows_b, sems_b)

            scatter_wait(dstb_a, rows_a, sems_a)
            load_dw(ka + 2, dstb_a, wb_a, semdw_a)
            wait_src(srcb_a, semsrc_a)
            gather(srcb_a, rows_a, semg_a)

            scatter_wait(dstb_b, rows_b, sems_b)
            load_dw(ka + 3, dstb_b, wb_b, semdw_b)
            wait_src(srcb_b, semsrc_b)
            gather(srcb_b, rows_b, semg_b)
            return carry
        lax.fori_loop(0, nch // 2 - 1, step, 0)

        gather_wait(srcb_a, rows_a, semg_a)
        wait_dw(dstb_a, wb_a, semdw_a)
        scale(wb_a, rows_a)
        scatter(dstb_a, rows_a, sems_a)
        gather_wait(srcb_b, rows_b, semg_b)
        wait_dw(dstb_b, wb_b, semdw_b)
        scale(wb_b, rows_b)
        scatter(dstb_b, rows_b, sems_b)
        scatter_wait(dstb_a, rows_a, sems_a)
        scatter_wait(dstb_b, rows_b, sems_b)
        plsc.subcore_barrier()

        for q in range((nrch + _NS - 1) // _NS):
            idx = sid + _NS * q

            @pl.when(idx < nrch)
            def _():
                r0 = idx * zr
                pltpu.sync_copy(acc.at[pl.ds(r0, zr)],
                                out_hbm.at[cid, h, pl.ds(r0, zr)])

        if h == 0:
            lax.fori_loop(0, zc, zrow, 0)
            plsc.subcore_barrier()


def _combine_body(p_ref, o_ref):
    half = o_ref.shape[-1] // 2
    o_ref[:, :half] = p_ref[0, 0] + p_ref[1, 0]
    o_ref[:, half:] = p_ref[0, 1] + p_ref[1, 1]


@jax.jit
def kernel(input, edge_index, edge_weight):
    n, d = input.shape
    e = edge_index.shape[1]
    assert e % _NW == 0 and d % (2 * _L) == 0 and n % 80 == 0
    e_t = e // _NW
    nch = -(-e_t // _C)
    nch += nch % 2
    e_pad = nch * _C
    half = d // 2

    pad = ((0, 0), (0, e_pad - e_t))
    shape4 = (_NW, nch, 1, _C)
    src = jnp.pad(edge_index[1].reshape(_NW, e_t), pad).reshape(shape4)
    dst = jnp.pad(edge_index[0].reshape(_NW, e_t), pad).reshape(shape4)
    w = jnp.pad(edge_weight.reshape(_NW, e_t), pad).reshape(shape4)
    x2 = jnp.stack([input[:, :half], input[:, half:]])

    zr = 200
    assert n % zr == 0 and zr % 8 == 0

    mesh = plsc.VectorSubcoreMesh(core_axis_name="c", subcore_axis_name="s",
                                  num_cores=_NC, num_subcores=_NS)
    partial = pl.kernel(
        functools.partial(_sc_body, n, d, nch, zr),
        out_type=jax.ShapeDtypeStruct((_NC, 2, n, half), jnp.float32),
        mesh=mesh,
        compiler_params=pltpu.CompilerParams(needs_layout_passes=False,
                                             use_tc_tiling_on_sc=False),
        scratch_types=[
            pltpu.MemorySpace.VMEM_SHARED((n, half), jnp.float32),
            pltpu.MemorySpace.VMEM_SHARED((n, half), jnp.float32),
            pltpu.VMEM((_C, half), jnp.float32),
            pltpu.VMEM((_C, half), jnp.float32),
            pltpu.VMEM((_C,), jnp.int32),
            pltpu.VMEM((_C,), jnp.int32),
            pltpu.VMEM((_C,), jnp.int32),
            pltpu.VMEM((_C,), jnp.int32),
            pltpu.VMEM((_C,), jnp.float32),
            pltpu.VMEM((_C,), jnp.float32),
            pltpu.SemaphoreType.DMA,
            pltpu.SemaphoreType.DMA,
            pltpu.SemaphoreType.DMA,
            pltpu.SemaphoreType.DMA,
            pltpu.SemaphoreType.DMA,
            pltpu.SemaphoreType.DMA,
            pltpu.SemaphoreType.DMA,
            pltpu.SemaphoreType.DMA,
        ],
    )(x2, src, dst, w)

    r = 2000
    return pl.pallas_call(
        _combine_body,
        grid=(n // r,),
        in_specs=[pl.BlockSpec((2, 2, r, half), lambda i: (0, 0, i, 0))],
        out_specs=pl.BlockSpec((r, d), lambda i: (i, 0)),
        out_shape=jax.ShapeDtypeStruct((n, d), jnp.float32),
    )(partial)
